# baseline (speedup 1.0000x reference)
import functools

import jax
import jax.numpy as jnp
from jax import lax
from jax.experimental import pallas as pl
from jax.experimental.pallas import tpu as pltpu
from jax.experimental.pallas import tpu_sc as plsc

_B = 16384
_D = 1024
_NT = 1000
_TPAD = 1024
_ROWS = 2048
_NB = _B // _ROWS

_NW = 32
_BPW = _B // _NW
_CHUNK = 128
_NCH = _BPW // _CHUNK

_sc_mesh = plsc.VectorSubcoreMesh(core_axis_name="c", subcore_axis_name="s")


@functools.partial(
    pl.kernel,
    mesh=_sc_mesh,
    out_type=jax.ShapeDtypeStruct((_B // _CHUNK, _CHUNK), jnp.float32),
    scratch_types=[
        pltpu.VMEM((_NCH, _CHUNK), jnp.int32),
        pltpu.VMEM((_NCH, _CHUNK), jnp.float32),
        pltpu.SemaphoreType.DMA,
    ],
)
def _sc_gather(tbl_hbm, idx_hbm, out_hbm, idx_v, ac_v, sem):
    wid = lax.axis_index("s") * 2 + lax.axis_index("c")
    row0 = wid * _NCH
    pltpu.sync_copy(idx_hbm.at[pl.ds(row0, _NCH)], idx_v)
    copies = [
        pltpu.async_copy(tbl_hbm.at[idx_v.at[j]], ac_v.at[j], sem)
        for j in range(_NCH)
    ]
    for c in copies:
        c.wait()
    pltpu.sync_copy(ac_v, out_hbm.at[pl.ds(row0, _NCH)])


def _dense_kernel(ac_ref, x0_ref, nz_ref, out_ref):
    ac = ac_ref[...]
    sa = jnp.sqrt(ac)
    sb = jnp.sqrt(1.0 - ac)
    out_ref[...] = sa * x0_ref[...] + sb * nz_ref[...]


@jax.jit
def kernel(x_0, timesteps, noise, alphas_cumprod):
    tbl = jnp.pad(alphas_cumprod, (0, _TPAD - _NT))
    idx = timesteps.reshape(_B // _CHUNK, _CHUNK)
    ac = _sc_gather(tbl, idx).reshape(_B, 1)
    return pl.pallas_call(
        _dense_kernel,
        grid=(_NB,),
        in_specs=[
            pl.BlockSpec((_ROWS, 1), lambda i: (i, 0)),
            pl.BlockSpec((_ROWS, _D), lambda i: (i, 0)),
            pl.BlockSpec((_ROWS, _D), lambda i: (i, 0)),
        ],
        out_specs=pl.BlockSpec((_ROWS, _D), lambda i: (i, 0)),
        out_shape=jax.ShapeDtypeStruct((_B, _D), jnp.float32),
        compiler_params=pltpu.CompilerParams(
            dimension_semantics=("arbitrary",),
        ),
    )(ac, x_0, noise)


# speedup vs baseline: 1.0094x; 1.0094x over previous
import functools

import jax
import jax.numpy as jnp
from jax import lax
from jax.experimental import pallas as pl
from jax.experimental.pallas import tpu as pltpu
from jax.experimental.pallas import tpu_sc as plsc

_B = 16384
_D = 1024
_NT = 1000
_TPAD = 1024
_ROWS = 1024
_NB = _B // _ROWS

_NW = 32
_BPW = _B // _NW
_CHUNK = 128
_NCH = _BPW // _CHUNK

_sc_mesh = plsc.VectorSubcoreMesh(core_axis_name="c", subcore_axis_name="s")


@functools.partial(
    pl.kernel,
    mesh=_sc_mesh,
    out_type=jax.ShapeDtypeStruct((_B // _CHUNK, _CHUNK), jnp.float32),
    scratch_types=[
        pltpu.VMEM((_NCH, _CHUNK), jnp.int32),
        pltpu.VMEM((_NCH, _CHUNK), jnp.float32),
        pltpu.SemaphoreType.DMA,
    ],
)
def _sc_gather(tbl_hbm, idx_hbm, out_hbm, idx_v, ac_v, sem):
    wid = lax.axis_index("s") * 2 + lax.axis_index("c")
    row0 = wid * _NCH
    pltpu.sync_copy(idx_hbm.at[pl.ds(row0, _NCH)], idx_v)
    copies = [
        pltpu.async_copy(tbl_hbm.at[idx_v.at[j]], ac_v.at[j], sem)
        for j in range(_NCH)
    ]
    for c in copies:
        c.wait()
    pltpu.sync_copy(ac_v, out_hbm.at[pl.ds(row0, _NCH)])


def _dense_kernel(ac_ref, x0_ref, nz_ref, out_ref):
    ac = ac_ref[...]
    sa = jnp.sqrt(ac)
    sb = jnp.sqrt(1.0 - ac)
    out_ref[...] = sa * x0_ref[...] + sb * nz_ref[...]


@jax.jit
def kernel(x_0, timesteps, noise, alphas_cumprod):
    idx = timesteps.reshape(_B // _CHUNK, _CHUNK)
    ac = _sc_gather(alphas_cumprod, idx).reshape(_B, 1)
    return pl.pallas_call(
        _dense_kernel,
        grid=(_NB,),
        in_specs=[
            pl.BlockSpec((_ROWS, 1), lambda i: (i, 0)),
            pl.BlockSpec((_ROWS, _D), lambda i: (i, 0)),
            pl.BlockSpec((_ROWS, _D), lambda i: (i, 0)),
        ],
        out_specs=pl.BlockSpec((_ROWS, _D), lambda i: (i, 0)),
        out_shape=jax.ShapeDtypeStruct((_B, _D), jnp.float32),
        compiler_params=pltpu.CompilerParams(
            dimension_semantics=("arbitrary",),
        ),
    )(ac, x_0, noise)


# speedup vs baseline: 1.3695x; 1.3567x over previous
import jax
import jax.numpy as jnp
from jax.experimental import pallas as pl
from jax.experimental.pallas import tpu as pltpu

_B = 16384
_D = 1024
_NT = 1000
_TPAD = 1024
_ROWS = 1024
_NB = _B // _ROWS


def _block_kernel(ts_ref, tbl_ref, x0_ref, nz_ref, out_ref):
    ts = ts_ref[...]
    tbl = tbl_ref[...]
    k = jax.lax.broadcasted_iota(jnp.int32, (_ROWS, _TPAD), 1)
    ac = jnp.sum(jnp.where(k == ts, tbl, 0.0), axis=1, keepdims=True)
    sa = jnp.sqrt(ac)
    sb = jnp.sqrt(1.0 - ac)
    out_ref[...] = sa * x0_ref[...] + sb * nz_ref[...]


@jax.jit
def kernel(x_0, timesteps, noise, alphas_cumprod):
    tbl = jnp.pad(alphas_cumprod, (0, _TPAD - _NT)).reshape(1, _TPAD)
    return pl.pallas_call(
        _block_kernel,
        grid=(_NB,),
        in_specs=[
            pl.BlockSpec((_ROWS, 1), lambda i: (i, 0)),
            pl.BlockSpec((1, _TPAD), lambda i: (0, 0)),
            pl.BlockSpec((_ROWS, _D), lambda i: (i, 0)),
            pl.BlockSpec((_ROWS, _D), lambda i: (i, 0)),
        ],
        out_specs=pl.BlockSpec((_ROWS, _D), lambda i: (i, 0)),
        out_shape=jax.ShapeDtypeStruct((_B, _D), jnp.float32),
        compiler_params=pltpu.CompilerParams(
            dimension_semantics=("arbitrary",),
        ),
    )(timesteps, tbl, x_0, noise)
